# Initial kernel scaffold; baseline (speedup 1.0000x reference)
import jax
import jax.numpy as jnp
from jax.experimental import pallas as pl


def kernel(x, batch, W_ih, W_hh, b_ih, b_hh):
    raise NotImplementedError("write your pallas kernel here")


# speedup vs baseline: 7.7045x; 7.7045x over previous
import functools

import jax
import jax.numpy as jnp
from jax.experimental import pallas as pl
from jax.experimental.pallas import tpu as pltpu

_D = 128
_B = 64
_T = 4
_BS = 2048


def _set2set_body(nblk, xT_ref, batch_ref, wih_ref, whh_ref, bias_ref, out_ref):
    D, B, T, BS = _D, _B, _T, _BS
    f32 = jnp.float32
    hi = jax.lax.Precision.HIGHEST

    h = jnp.zeros((B, D), f32)
    c = jnp.zeros((B, D), f32)
    q_star = jnp.zeros((B, 2 * D), f32)
    seg_ids = jax.lax.broadcasted_iota(jnp.int32, (B, 1), 0)

    for _ in range(T):
        gates = (
            jnp.dot(q_star, wih_ref[...], precision=hi, preferred_element_type=f32)
            + jnp.dot(h, whh_ref[...], precision=hi, preferred_element_type=f32)
            + bias_ref[...]
        )
        ig = jax.nn.sigmoid(gates[:, :D])
        fg = jax.nn.sigmoid(gates[:, D : 2 * D])
        gg = jnp.tanh(gates[:, 2 * D : 3 * D])
        og = jax.nn.sigmoid(gates[:, 3 * D :])
        c = fg * c + ig * gg
        h = og * jnp.tanh(c)
        q = h

        def blk(j, carry):
            m, den, rnum = carry
            xbT = xT_ref[:, pl.ds(j * BS, BS)]
            bb = batch_ref[j]
            mask = bb == seg_ids
            ET = jax.lax.dot_general(
                q, xbT, (((1,), (0,)), ((), ())), precision=hi,
                preferred_element_type=f32,
            )
            m_part = jnp.max(jnp.where(mask, ET, -jnp.inf), axis=1, keepdims=True)
            m_new = jnp.maximum(m, m_part)
            scale = jnp.exp(m - m_new)
            P = jnp.where(mask, jnp.exp(ET - m_new), 0.0)
            den = den * scale + jnp.sum(P, axis=1, keepdims=True)
            rnum = rnum * scale + jax.lax.dot_general(
                P, xbT, (((1,), (1,)), ((), ())), precision=hi,
                preferred_element_type=f32,
            )
            return m_new, den, rnum

        m0 = jnp.full((B, 1), -1e30, f32)
        m, den, rnum = jax.lax.fori_loop(
            0, nblk, blk, (m0, jnp.zeros((B, 1), f32), jnp.zeros((B, D), f32))
        )
        r = rnum / jnp.maximum(den, 1e-30)
        q_star = jnp.concatenate([q, r], axis=1)

    out_ref[...] = q_star


def kernel(x, batch, W_ih, W_hh, b_ih, b_hh):
    n, d = x.shape
    assert d == _D
    nblk = -(-n // _BS)
    np_ = nblk * _BS
    xT = jnp.pad(x, ((0, np_ - n), (0, 0))).T
    batch3 = jnp.pad(batch, (0, np_ - n), constant_values=_B).reshape(nblk, 1, _BS)
    wihT = W_ih.T
    whhT = W_hh.T
    bias = (b_ih + b_hh).reshape(1, 4 * _D)

    return pl.pallas_call(
        functools.partial(_set2set_body, nblk),
        out_shape=jax.ShapeDtypeStruct((_B, 2 * _D), x.dtype),
        compiler_params=pltpu.CompilerParams(
            vmem_limit_bytes=64 * 1024 * 1024,
        ),
    )(xT, batch3, wihT, whhT, bias)


# speedup vs baseline: 7.7715x; 1.0087x over previous
import functools

import jax
import jax.numpy as jnp
from jax.experimental import pallas as pl
from jax.experimental.pallas import tpu as pltpu

_D = 128
_B = 64
_T = 4
_BS = 2000


def _set2set_body(nblk, x_ref, batch_ref, wih_ref, whh_ref, bias_ref, out_ref):
    D, B, T, BS = _D, _B, _T, _BS
    f32 = jnp.float32
    hi = jax.lax.Precision.HIGHEST

    h = jnp.zeros((B, D), f32)
    c = jnp.zeros((B, D), f32)
    q_star = jnp.zeros((B, 2 * D), f32)
    seg_ids = jax.lax.broadcasted_iota(jnp.int32, (B, 1), 0)

    for _ in range(T):
        gates = (
            jnp.dot(q_star, wih_ref[...], precision=hi, preferred_element_type=f32)
            + jnp.dot(h, whh_ref[...], precision=hi, preferred_element_type=f32)
            + bias_ref[...]
        )
        ig = jax.nn.sigmoid(gates[:, :D])
        fg = jax.nn.sigmoid(gates[:, D : 2 * D])
        gg = jnp.tanh(gates[:, 2 * D : 3 * D])
        og = jax.nn.sigmoid(gates[:, 3 * D :])
        c = fg * c + ig * gg
        h = og * jnp.tanh(c)
        q = h

        def blk(j, carry):
            m, den, rnum = carry
            xb = x_ref[pl.ds(j * BS, BS), :]
            bb = batch_ref[j]
            mask = bb == seg_ids
            ET = jax.lax.dot_general(
                q, xb, (((1,), (1,)), ((), ())), precision=hi,
                preferred_element_type=f32,
            )
            m_part = jnp.max(jnp.where(mask, ET, -jnp.inf), axis=1, keepdims=True)
            m_new = jnp.maximum(m, m_part)
            scale = jnp.exp(m - m_new)
            P = jnp.where(mask, jnp.exp(ET - m_new), 0.0)
            den = den * scale + jnp.sum(P, axis=1, keepdims=True)
            rnum = rnum * scale + jax.lax.dot_general(
                P, xb, (((1,), (0,)), ((), ())), precision=hi,
                preferred_element_type=f32,
            )
            return m_new, den, rnum

        m0 = jnp.full((B, 1), -1e30, f32)
        m, den, rnum = jax.lax.fori_loop(
            0, nblk, blk, (m0, jnp.zeros((B, 1), f32), jnp.zeros((B, D), f32))
        )
        r = rnum / jnp.maximum(den, 1e-30)
        q_star = jnp.concatenate([q, r], axis=1)

    out_ref[...] = q_star


def kernel(x, batch, W_ih, W_hh, b_ih, b_hh):
    n, d = x.shape
    assert d == _D and n % _BS == 0
    nblk = n // _BS
    batch3 = batch.reshape(nblk, 1, _BS)
    wihT = W_ih.T
    whhT = W_hh.T
    bias = (b_ih + b_hh).reshape(1, 4 * _D)

    return pl.pallas_call(
        functools.partial(_set2set_body, nblk),
        out_shape=jax.ShapeDtypeStruct((_B, 2 * _D), x.dtype),
        compiler_params=pltpu.CompilerParams(
            vmem_limit_bytes=64 * 1024 * 1024,
        ),
    )(x, batch3, wihT, whhT, bias)


# speedup vs baseline: 9.0766x; 1.1679x over previous
import functools

import jax
import jax.numpy as jnp
from jax.experimental import pallas as pl
from jax.experimental.pallas import tpu as pltpu

_D = 128
_B = 64
_T = 4
_BS = 4000


def _set2set_body(nblk, x_ref, batch_ref, wih_ref, whh_ref, bias_ref, out_ref):
    D, B, T, BS = _D, _B, _T, _BS
    f32 = jnp.float32
    hi = jax.lax.Precision.HIGHEST

    h = jnp.zeros((B, D), f32)
    c = jnp.zeros((B, D), f32)
    q_star = jnp.zeros((B, 2 * D), f32)
    seg_ids = jax.lax.broadcasted_iota(jnp.int32, (B, 1), 0)

    for _ in range(T):
        gates = (
            jnp.dot(q_star, wih_ref[...], precision=hi, preferred_element_type=f32)
            + jnp.dot(h, whh_ref[...], precision=hi, preferred_element_type=f32)
            + bias_ref[...]
        )
        ig = jax.nn.sigmoid(gates[:, :D])
        fg = jax.nn.sigmoid(gates[:, D : 2 * D])
        gg = jnp.tanh(gates[:, 2 * D : 3 * D])
        og = jax.nn.sigmoid(gates[:, 3 * D :])
        c = fg * c + ig * gg
        h = og * jnp.tanh(c)
        q = h

        def blk(j, carry):
            m, den, rnum = carry
            xb = x_ref[pl.ds(j * BS, BS), :]
            bb = batch_ref[j]
            mbias = jnp.where(bb == seg_ids, 0.0, -jnp.inf)
            ET = jax.lax.dot_general(
                q, xb, (((1,), (1,)), ((), ())), precision=hi,
                preferred_element_type=f32,
            )
            Em = ET + mbias
            m_part = jnp.max(Em, axis=1, keepdims=True)
            m_new = jnp.maximum(m, m_part)
            scale = jnp.exp(m - m_new)
            P = jnp.exp(Em - m_new)

            den = den * scale + jnp.sum(P, axis=1, keepdims=True)
            rnum = rnum * scale + jax.lax.dot_general(
                P, xb, (((1,), (0,)), ((), ())), precision=hi,
                preferred_element_type=f32,
            )
            return m_new, den, rnum

        m0 = jnp.full((B, 1), -1e30, f32)
        m, den, rnum = jax.lax.fori_loop(
            0, nblk, blk, (m0, jnp.zeros((B, 1), f32), jnp.zeros((B, D), f32))
        )
        r = rnum / jnp.maximum(den, 1e-30)
        q_star = jnp.concatenate([q, r], axis=1)

    out_ref[...] = q_star


def kernel(x, batch, W_ih, W_hh, b_ih, b_hh):
    n, d = x.shape
    assert d == _D and n % _BS == 0
    nblk = n // _BS
    batch3 = batch.reshape(nblk, 1, _BS)
    wihT = W_ih.T
    whhT = W_hh.T
    bias = (b_ih + b_hh).reshape(1, 4 * _D)

    return pl.pallas_call(
        functools.partial(_set2set_body, nblk),
        out_shape=jax.ShapeDtypeStruct((_B, 2 * _D), x.dtype),
        compiler_params=pltpu.CompilerParams(
            vmem_limit_bytes=64 * 1024 * 1024,
        ),
    )(x, batch3, wihT, whhT, bias)
